# baseline (speedup 1.0000x reference)
import functools

import jax
import jax.numpy as jnp
from jax import lax
from jax.experimental import pallas as pl
from jax.experimental.pallas import tpu as pltpu
from jax.experimental.pallas import tpu_sc as plsc

NUM_KEYS = 13
EMBED_DIM = 64
BATCH = 16384

NC, NS = 2, 16
NW = NC * NS
BPW = BATCH // NW
LANES = 16
GROUPS = BPW // LANES
BLOCK_ROWS = 128
NBLOCKS = BPW // BLOCK_ROWS
GPB = BLOCK_ROWS // LANES


def _projt_body(emb_ref, w_ref, b_ref, out_ref):
    out_ref[...] = lax.dot_general(
        w_ref[...], emb_ref[...],
        dimension_numbers=(((1,), (1,)), ((), ())),
        preferred_element_type=jnp.float32,
    ) + b_ref[...]


def _project_table_t(emb_table, W, b):
    emb_pad = jnp.pad(emb_table, ((0, LANES - NUM_KEYS), (0, 0)))
    return pl.pallas_call(
        _projt_body,
        out_shape=jax.ShapeDtypeStruct((EMBED_DIM, LANES), jnp.float32),
    )(emb_pad, W, b.reshape(EMBED_DIM, 1))


def _gather_body(tabt_hbm, idx_hbm, out_hbm, tabt_v, idx_v, rows_v, sem):
    wid = lax.axis_index("s") * NC + lax.axis_index("c")
    pltpu.async_copy(tabt_hbm, tabt_v, sem).wait()
    pltpu.sync_copy(idx_hbm.at[pl.ds(wid * BPW, BPW)], idx_v)

    row_offs = lax.iota(jnp.int32, LANES) * EMBED_DIM
    dnums = lax.GatherDimensionNumbers(
        offset_dims=(), collapsed_slice_dims=(0,), start_index_map=(0,))

    def group(g, carry):
        vi = idx_v[pl.ds(g * LANES, LANES)]
        obase = row_offs + g * (LANES * EMBED_DIM)
        for d in range(EMBED_DIM):
            vals = lax.gather(
                tabt_v[d], vi[:, None], dnums, (1,),
                mode=lax.GatherScatterMode.PROMISE_IN_BOUNDS)
            plsc.store_scatter(rows_v, [obase + d], vals)
        return carry

    copies = []
    for blk in range(NBLOCKS):
        lax.fori_loop(blk * GPB, (blk + 1) * GPB, group, 0)
        copies.append(pltpu.async_copy(
            rows_v.at[pl.ds(blk * BLOCK_ROWS * EMBED_DIM,
                            BLOCK_ROWS * EMBED_DIM)],
            out_hbm.at[pl.ds((wid * BPW + blk * BLOCK_ROWS) * EMBED_DIM,
                             BLOCK_ROWS * EMBED_DIM)],
            sem,
        ))
    for c in copies:
        c.wait()


_gather = pl.kernel(
    _gather_body,
    out_type=jax.ShapeDtypeStruct((BATCH * EMBED_DIM,), jnp.float32),
    mesh=plsc.VectorSubcoreMesh(
        core_axis_name="c", subcore_axis_name="s",
        num_cores=NC, num_subcores=NS,
    ),
    scratch_types=[
        pltpu.VMEM((EMBED_DIM, LANES), jnp.float32),
        pltpu.VMEM((BPW,), jnp.int32),
        pltpu.VMEM((BPW * EMBED_DIM,), jnp.float32),
        pltpu.SemaphoreType.DMA,
    ],
    compiler_params=pltpu.CompilerParams(
        use_tc_tiling_on_sc=False, needs_layout_passes=False),
)


def kernel(key_idx, emb_table, W, b):
    projt = _project_table_t(emb_table, W, b)
    idx = key_idx.astype(jnp.int32)
    out = _gather(projt, idx)
    return out.reshape(BATCH, 1, EMBED_DIM)


# speedup vs baseline: 1.2010x; 1.2010x over previous
import functools

import jax
import jax.numpy as jnp
from jax import lax
from jax.experimental import pallas as pl
from jax.experimental.pallas import tpu as pltpu
from jax.experimental.pallas import tpu_sc as plsc

NUM_KEYS = 13
EMBED_DIM = 64
BATCH = 16384

NC, NS = 2, 16
NW = NC * NS
BPW = BATCH // NW
LANES = 16
GROUPS = BPW // LANES
PSTRIDE = EMBED_DIM + 1
BLOCK_ROWS = 128
NBLOCKS = BPW // BLOCK_ROWS
GPB = BLOCK_ROWS // LANES


def _projt_body(emb_ref, w_ref, b_ref, out_ref):
    out_ref[...] = lax.dot_general(
        w_ref[...], emb_ref[...],
        dimension_numbers=(((1,), (1,)), ((), ())),
        preferred_element_type=jnp.float32,
    ) + b_ref[...]


def _project_table_t(emb_table, W, b):
    emb_pad = jnp.pad(emb_table, ((0, LANES - NUM_KEYS), (0, 0)))
    return pl.pallas_call(
        _projt_body,
        out_shape=jax.ShapeDtypeStruct((EMBED_DIM, LANES), jnp.float32),
    )(emb_pad, W, b.reshape(EMBED_DIM, 1))


def _gather_body(tabt_hbm, idx_hbm, out_hbm, tabt_v, idx_v, rows_v, sem):
    wid = lax.axis_index("s") * NC + lax.axis_index("c")
    pltpu.async_copy(tabt_hbm, tabt_v, sem).wait()
    pltpu.sync_copy(idx_hbm.at[pl.ds(wid * BPW, BPW)], idx_v)

    lane_iota = lax.iota(jnp.int32, LANES)
    dnums = lax.GatherDimensionNumbers(
        offset_dims=(), collapsed_slice_dims=(0,), start_index_map=(0,))
    col_consts = [jnp.full((LANES,), d, jnp.int32) for d in range(EMBED_DIM)]

    def group(g, carry):
        vi = idx_v[pl.ds(g * LANES, LANES)]
        rid = lane_iota + g * LANES
        for d in range(EMBED_DIM):
            vals = lax.gather(
                tabt_v[d], vi[:, None], dnums, (1,),
                mode=lax.GatherScatterMode.PROMISE_IN_BOUNDS)
            plsc.store_scatter(rows_v, [rid, col_consts[d]], vals)
        return carry

    copies = []
    for blk in range(NBLOCKS):
        lax.fori_loop(blk * GPB, (blk + 1) * GPB, group, 0)
        copies.append(pltpu.async_copy(
            rows_v.at[pl.ds(blk * BLOCK_ROWS, BLOCK_ROWS),
                      pl.ds(0, EMBED_DIM)],
            out_hbm.at[pl.ds(wid * BPW + blk * BLOCK_ROWS, BLOCK_ROWS)],
            sem,
        ))
    for c in copies:
        c.wait()


_gather = pl.kernel(
    _gather_body,
    out_type=jax.ShapeDtypeStruct((BATCH, EMBED_DIM), jnp.float32),
    mesh=plsc.VectorSubcoreMesh(
        core_axis_name="c", subcore_axis_name="s",
        num_cores=NC, num_subcores=NS,
    ),
    scratch_types=[
        pltpu.VMEM((EMBED_DIM, LANES), jnp.float32),
        pltpu.VMEM((BPW,), jnp.int32),
        pltpu.VMEM((BPW, PSTRIDE), jnp.float32),
        pltpu.SemaphoreType.DMA,
    ],
    compiler_params=pltpu.CompilerParams(
        use_tc_tiling_on_sc=False, needs_layout_passes=False),
)


def kernel(key_idx, emb_table, W, b):
    projt = _project_table_t(emb_table, W, b)
    idx = key_idx.astype(jnp.int32)
    out = _gather(projt, idx)
    return out[:, None, :]


# speedup vs baseline: 2.6015x; 2.1662x over previous
import functools

import jax
import jax.numpy as jnp
from jax import lax
from jax.experimental import pallas as pl
from jax.experimental.pallas import tpu as pltpu
from jax.experimental.pallas import tpu_sc as plsc

NUM_KEYS = 13
EMBED_DIM = 64
BATCH = 16384

NC, NS = 2, 16
NW = NC * NS
BPW = BATCH // NW
LANES = 16
GROUPS = BPW // LANES
DBLOCK = 32
NDBLK = EMBED_DIM // DBLOCK


def _projt_body(emb_ref, w_ref, b_ref, out_ref):
    res = lax.dot_general(
        w_ref[...], emb_ref[...],
        dimension_numbers=(((1,), (1,)), ((), ())),
        preferred_element_type=jnp.float32,
    ) + b_ref[...].T
    out_ref[:, :NUM_KEYS] = res
    out_ref[:, NUM_KEYS:] = jnp.zeros(
        (EMBED_DIM, LANES - NUM_KEYS), jnp.float32)


def _project_table_t(emb_table, W, b):
    return pl.pallas_call(
        _projt_body,
        out_shape=jax.ShapeDtypeStruct((EMBED_DIM, LANES), jnp.float32),
    )(emb_table, W, b.reshape(1, EMBED_DIM))


def _gather_body(tabt_hbm, idx_hbm, out_hbm, tabt_v, idx_v, cols_v, sem):
    wid = lax.axis_index("s") * NC + lax.axis_index("c")
    pltpu.async_copy(tabt_hbm, tabt_v, sem).wait()
    pltpu.sync_copy(idx_hbm.at[pl.ds(wid * BPW, BPW)], idx_v)

    dnums = lax.GatherDimensionNumbers(
        offset_dims=(), collapsed_slice_dims=(0,), start_index_map=(0,))

    def make_group(blk, colregs):
        def group(g, carry):
            vi = idx_v[pl.ds(g * LANES, LANES)]
            for dd in range(DBLOCK):
                vals = lax.gather(
                    colregs[dd], vi[:, None], dnums, (1,),
                    mode=lax.GatherScatterMode.PROMISE_IN_BOUNDS)
                cols_v[blk * DBLOCK + dd, pl.ds(g * LANES, LANES)] = vals
            return carry
        return group

    copies = []
    for blk in range(NDBLK):
        colregs = [tabt_v[blk * DBLOCK + dd] for dd in range(DBLOCK)]
        lax.fori_loop(0, GROUPS, make_group(blk, colregs), 0)
        copies.append(pltpu.async_copy(
            cols_v.at[pl.ds(blk * DBLOCK, DBLOCK)],
            out_hbm.at[pl.ds(blk * DBLOCK, DBLOCK),
                       pl.ds(wid * BPW, BPW)],
            sem,
        ))
    for c in copies:
        c.wait()


_gather = pl.kernel(
    _gather_body,
    out_type=jax.ShapeDtypeStruct((EMBED_DIM, BATCH), jnp.float32),
    mesh=plsc.VectorSubcoreMesh(
        core_axis_name="c", subcore_axis_name="s",
        num_cores=NC, num_subcores=NS,
    ),
    scratch_types=[
        pltpu.VMEM((EMBED_DIM, LANES), jnp.float32),
        pltpu.VMEM((BPW,), jnp.int32),
        pltpu.VMEM((EMBED_DIM, BPW), jnp.float32),
        pltpu.SemaphoreType.DMA,
    ],
    compiler_params=pltpu.CompilerParams(
        use_tc_tiling_on_sc=True, needs_layout_passes=False),
)


def kernel(key_idx, emb_table, W, b):
    projt = _project_table_t(emb_table, W, b)
    idx = key_idx.astype(jnp.int32)
    out_t = _gather(projt, idx)
    return out_t.T[:, None, :]
